# baseline (speedup 1.0000x reference)
import itertools

import jax
import jax.numpy as jnp
import numpy as np
from jax.experimental import pallas as pl
from jax.experimental.pallas import tpu as pltpu
from jax.experimental.pallas import tpu_sc as plsc

B = 64
N = 32
R = 16
F = 16
G = 4960

OCT = 8
NOCT = B // OCT
ROWS = NOCT * G
W = 128
ROWS_PAD = 40960
NWIN = ROWS_PAD // W


def _build_static_indices():
    groups = np.array(list(itertools.combinations(range(N), 3)), dtype=np.int64)
    a0, a1, a2 = groups[:, 0], groups[:, 1], groups[:, 2]
    i1 = a0 * N + a1
    i2 = N * N + a0 * N + a2
    i3 = 2 * N * N + a1 * N + a2
    boff = (np.arange(NOCT, dtype=np.int64) * (3 * N * N))[:, None]

    def flat(i):
        v = (boff + i[None, :]).reshape(-1).astype(np.int32)
        return np.pad(v, (0, ROWS_PAD - ROWS)).reshape(1, -1)

    return flat(i1), flat(i2), flat(i3)


_I1, _I2, _I3 = _build_static_indices()


def _stage1_body(x_ref, w_ref, o_ref):
    y = jnp.dot(x_ref[0], w_ref[...], preferred_element_type=jnp.float32)
    o_ref[0, 0] = y[:, 0:16]
    o_ref[0, 1] = y[:, 16:32]
    o_ref[0, 2] = y[:, 32:48]


def _stage1(xa, wa):
    return pl.pallas_call(
        _stage1_body,
        grid=(B,),
        in_specs=[
            pl.BlockSpec((1, N * N, 4 * R), lambda b: (b, 0, 0)),
            pl.BlockSpec((4 * R, 3 * F), lambda b: (0, 0)),
        ],
        out_specs=pl.BlockSpec((1, 3, N * N, F), lambda b: (b, 0, 0, 0)),
        out_shape=jax.ShapeDtypeStruct((B, 3, N * N, F), jnp.float32),
    )(xa, wa)


def _sc_gather_sum(t8, i1, i2, i3):
    mesh = plsc.VectorSubcoreMesh(core_axis_name="c", subcore_axis_name="s")

    def body(t_hbm, i1_hbm, i2_hbm, i3_hbm, o_hbm, r2, r3, sem2, sem3):
        def window(i1b, i2b, i3b, ob):
            c2 = pltpu.async_copy(t_hbm.at[i2b.at[0]], r2, sem2)
            c3 = pltpu.async_copy(t_hbm.at[i3b.at[0]], r3, sem3)
            pltpu.sync_copy(t_hbm.at[i1b.at[0]], ob)
            c2.wait()
            c3.wait()

            @pl.loop(0, W)
            def _row(r):
                for j in range(OCT):
                    sl = pl.ds(j * F, F)
                    ob[r, sl] = ob[r, sl] + r2[r, sl] + r3[r, sl]

        pltpu.emit_pipeline(
            window,
            grid=(NWIN,),
            in_specs=[
                pl.BlockSpec((1, W), lambda i: (0, i)),
                pl.BlockSpec((1, W), lambda i: (0, i)),
                pl.BlockSpec((1, W), lambda i: (0, i)),
            ],
            out_specs=[pl.BlockSpec((W, OCT * F), lambda i: (i, 0))],
            core_axis_name=("c", "s"),
            dimension_semantics=(pltpu.PARALLEL,),
        )(i1_hbm, i2_hbm, i3_hbm, o_hbm)

    sc_kernel = pl.kernel(
        body,
        out_type=jax.ShapeDtypeStruct((ROWS_PAD, OCT * F), jnp.float32),
        mesh=mesh,
        scratch_types=[
            pltpu.VMEM((W, OCT * F), jnp.float32),
            pltpu.VMEM((W, OCT * F), jnp.float32),
            pltpu.SemaphoreType.DMA,
            pltpu.SemaphoreType.DMA,
        ],
    )
    return sc_kernel(t8, i1, i2, i3)


def kernel(inputs, filters):
    idx = jnp.arange(N)
    in_t = jnp.swapaxes(inputs, 1, 2)
    diag = inputs[:, idx, idx, :]
    d_v = jnp.broadcast_to(diag[:, None, :, :], (B, N, N, R))
    d_u = jnp.broadcast_to(diag[:, :, None, :], (B, N, N, R))
    xa = jnp.concatenate([inputs, in_t, d_v, d_u], axis=-1).reshape(B, N * N, 4 * R)

    def fpq(p, q):
        return filters[:, p, q, :].T

    z = jnp.zeros((R, F), jnp.float32)
    wa01 = jnp.concatenate([fpq(0, 1), fpq(1, 0), fpq(1, 1), z], axis=0)
    wa02 = jnp.concatenate([fpq(0, 2), fpq(2, 0), z, fpq(0, 0)], axis=0)
    wa12 = jnp.concatenate([fpq(1, 2), fpq(2, 1), fpq(2, 2), z], axis=0)
    wa = jnp.concatenate([wa01, wa02, wa12], axis=1)

    tables = _stage1(xa, wa)
    t8 = (
        tables.reshape(NOCT, OCT, 3, N * N, F)
        .transpose(0, 2, 3, 1, 4)
        .reshape(NOCT * 3 * N * N, OCT * F)
    )

    res = _sc_gather_sum(t8, jnp.asarray(_I1), jnp.asarray(_I2), jnp.asarray(_I3))
    out = res[:ROWS].reshape(NOCT, G, OCT, F).transpose(0, 2, 1, 3).reshape(B, G, F)
    return out


# speedup vs baseline: 3.7495x; 3.7495x over previous
import itertools

import jax
import jax.numpy as jnp
import numpy as np
from jax.experimental import pallas as pl
from jax.experimental.pallas import tpu as pltpu
from jax.experimental.pallas import tpu_sc as plsc

B = 64
N = 32
R = 16
F = 16
G = 4960

OCT = 8
NOCT = B // OCT


def _fused_body(x_ref, w_ref, o_ref, scr_ref):
    y = jnp.dot(x_ref[0], w_ref[...], preferred_element_type=jnp.float32)
    scr_ref[0] = y[:, 0:128]
    scr_ref[1] = y[:, 128:256]
    scr_ref[2] = y[:, 256:384]
    off = 0
    for a in range(N - 2):
        for b2 in range(a + 1, N - 1):
            L = (N - 1) - b2
            r01 = scr_ref[0, a * N + b2, :]
            s02 = scr_ref[1, pl.ds(a * N + b2 + 1, L), :]
            s12 = scr_ref[2, pl.ds(b2 * N + b2 + 1, L), :]
            o_ref[0, pl.ds(off, L), :] = r01[None, :] + s02 + s12
            off += L


def _fused_tc(xab, w8):
    noct = xab.shape[0]
    return pl.pallas_call(
        _fused_body,
        grid=(noct,),
        in_specs=[
            pl.BlockSpec((1, N * N, 4 * R * OCT), lambda i: (i, 0, 0)),
            pl.BlockSpec((4 * R * OCT, 3 * OCT * F), lambda i: (0, 0)),
        ],
        out_specs=pl.BlockSpec((1, G, OCT * F), lambda i: (i, 0, 0)),
        out_shape=jax.ShapeDtypeStruct((noct, G, OCT * F), jnp.float32),
        scratch_shapes=[pltpu.VMEM((3, N * N, OCT * F), jnp.float32)],
        compiler_params=pltpu.CompilerParams(
            dimension_semantics=("parallel",),
        ),
    )(xab, w8)


def kernel(inputs, filters):
    idx = jnp.arange(N)
    in_t = jnp.swapaxes(inputs, 1, 2)
    diag = inputs[:, idx, idx, :]
    d_v = jnp.broadcast_to(diag[:, None, :, :], (B, N, N, R))
    d_u = jnp.broadcast_to(diag[:, :, None, :], (B, N, N, R))
    comp = jnp.concatenate([inputs, in_t, d_v, d_u], axis=-1)
    xab = (
        comp.reshape(NOCT, OCT, N * N, 4 * R)
        .transpose(0, 2, 3, 1)
        .reshape(NOCT, N * N, 4 * R * OCT)
    )

    def fpq(p, q):
        return filters[:, p, q, :].T

    z = jnp.zeros((R, F), jnp.float32)
    wa01 = jnp.concatenate([fpq(0, 1), fpq(1, 0), fpq(1, 1), z], axis=0)
    wa02 = jnp.concatenate([fpq(0, 2), fpq(2, 0), z, fpq(0, 0)], axis=0)
    wa12 = jnp.concatenate([fpq(1, 2), fpq(2, 1), fpq(2, 2), z], axis=0)
    wa3 = jnp.stack([wa01, wa02, wa12], axis=1)
    w8 = jnp.einsum("kcf,bj->kbcjf", wa3, jnp.eye(OCT, dtype=jnp.float32))
    w8 = w8.reshape(4 * R * OCT, 3 * OCT * F)

    res = _fused_tc(xab, w8)
    out = res.reshape(NOCT, G, OCT, F).transpose(0, 2, 1, 3).reshape(B, G, F)
    return out
